# baseline (speedup 1.0000x reference)
import functools

import jax
import jax.numpy as jnp
from jax import lax
from jax.experimental import pallas as pl
from jax.experimental.pallas import tpu as pltpu
from jax.experimental.pallas import tpu_sc as plsc

_B, _CTX, _D, _V = 4096, 20, 64, 100000
_DP = 128
_BH = _B // 2
_NC, _NS = 2, 16
_NW = _NC * _NS
_BPW = _BH // _NW
_GRP = 32
_NGRP = _BPW // _GRP

_mesh = plsc.VectorSubcoreMesh(core_axis_name="c", subcore_axis_name="s")


def _make_pool(half):
    @functools.partial(
        pl.kernel,
        mesh=_mesh,
        out_type=jax.ShapeDtypeStruct((_BH, _DP), jnp.float32),
        scratch_types=[
            pltpu.VMEM((_CTX, _BPW), jnp.int32),
            pltpu.VMEM((2, _CTX * _GRP, _D), jnp.float32),
            pltpu.VMEM((_BPW, _DP), jnp.float32),
            pltpu.SemaphoreType.DMA,
            pltpu.SemaphoreType.DMA,
        ],
        compiler_params=pltpu.CompilerParams(use_tc_tiling_on_sc=False),
    )
    def _pool(idx_hbm, table_hbm, out_hbm, idx20_v, rows_v, pooled_v,
              sem0, sem1):
        wid = lax.axis_index("s") * _NC + lax.axis_index("c")
        base = wid * _BPW
        for c in range(_CTX):
            pltpu.sync_copy(
                idx_hbm.at[pl.ds(
                    pl.multiple_of(c * _B + half * _BH + base, 8), _BPW)],
                idx20_v.at[c])

        sems = (sem0, sem1)
        zero = jnp.zeros((16,), jnp.float32)

        def fire(g, slot):
            off = pl.multiple_of(g * _GRP, 8)
            for c in range(_CTX):
                pltpu.async_copy(
                    table_hbm.at[idx20_v.at[c, pl.ds(off, _GRP)]],
                    rows_v.at[slot, pl.ds(c * _GRP, _GRP)],
                    sems[slot])

        def drain(slot):
            pltpu.make_async_copy(
                table_hbm.at[pl.ds(0, _CTX * _GRP), :],
                rows_v.at[slot],
                sems[slot]).wait()

        fire(0, 0)
        fire(1, 1)

        for g in range(_NGRP):
            b = g % 2
            drain(b)

            def row_body(i, acc_carry, b=b, g=g):
                row = g * _GRP + i
                for k in range(_D // 16):
                    acc = rows_v[b, i, pl.ds(k * 16, 16)]
                    for c in range(1, _CTX):
                        acc = acc + rows_v[b, c * _GRP + i, pl.ds(k * 16, 16)]
                    pooled_v[row, pl.ds(k * 16, 16)] = acc * (1.0 / _CTX)
                for k in range(_D // 16, _DP // 16):
                    pooled_v[row, pl.ds(k * 16, 16)] = zero
                return acc_carry

            lax.fori_loop(0, _GRP, row_body, 0)

        pltpu.sync_copy(pooled_v,
                        out_hbm.at[pl.ds(pl.multiple_of(base, 8), _BPW)])

    return _pool


_pool_h0 = _make_pool(0)
_pool_h1 = _make_pool(1)


_VB = 1024


def _proj_body(p_ref, w_ref, b_ref, o_ref):
    ot = lax.dot_general(
        w_ref[...], p_ref[...][:, :_D], (((0,), (1,)), ((), ())),
        preferred_element_type=jnp.float32,
    )
    o_ref[...] = ot + b_ref[...].T


def _prev_body(prev_ref, p_ref, w_ref, b_ref, o_ref):
    _proj_body(p_ref, w_ref, b_ref, o_ref)


def _project_h0(pooled, w_t, bias2d):
    return pl.pallas_call(
        _proj_body,
        grid=(pl.cdiv(_V, _VB),),
        in_specs=[
            pl.BlockSpec((_BH, _DP), lambda v: (0, 0)),
            pl.BlockSpec((_D, _VB), lambda v: (0, v)),
            pl.BlockSpec((1, _VB), lambda v: (0, v)),
        ],
        out_specs=pl.BlockSpec((_VB, _BH), lambda v: (v, 0)),
        out_shape=jax.ShapeDtypeStruct((_V, _B), jnp.float32),
    )(pooled, w_t, bias2d)


def _project_h1(prev, pooled, w_t, bias2d):
    return pl.pallas_call(
        _prev_body,
        grid=(pl.cdiv(_V, _VB),),
        in_specs=[
            pl.BlockSpec(memory_space=pl.ANY),
            pl.BlockSpec((_BH, _DP), lambda v: (0, 0)),
            pl.BlockSpec((_D, _VB), lambda v: (0, v)),
            pl.BlockSpec((1, _VB), lambda v: (0, v)),
        ],
        out_specs=pl.BlockSpec((_VB, _BH), lambda v: (v, 1)),
        out_shape=jax.ShapeDtypeStruct((_V, _B), jnp.float32),
        input_output_aliases={0: 0},
    )(prev, pooled, w_t, bias2d)


@jax.jit
def _forward(idx_flat, embeddings, w_t, bias2d):
    pooled0 = _pool_h0(idx_flat, embeddings)
    pooled1 = _pool_h1(idx_flat, embeddings)
    out_t = _project_h0(pooled0, w_t, bias2d)
    out_t = _project_h1(out_t, pooled1, w_t, bias2d)
    return out_t


def kernel(context_words_indices, embeddings, linear_w, linear_b):
    idx_flat = jnp.asarray(context_words_indices, jnp.int32).T.reshape(-1)
    out_t = _forward(idx_flat, embeddings, linear_w.T,
                     linear_b.reshape(1, _V))
    return out_t.T


# speedup vs baseline: 1.0152x; 1.0152x over previous
import functools

import jax
import jax.numpy as jnp
from jax import lax
from jax.experimental import pallas as pl
from jax.experimental.pallas import tpu as pltpu
from jax.experimental.pallas import tpu_sc as plsc

_B, _CTX, _D, _V = 4096, 20, 64, 100000
_DP = 128
_NC, _NS = 2, 16
_NW = _NC * _NS
_BPW = _B // _NW
_GRP = 32
_NGRP = _BPW // _GRP

_mesh = plsc.VectorSubcoreMesh(core_axis_name="c", subcore_axis_name="s")


@functools.partial(
    pl.kernel,
    mesh=_mesh,
    out_type=jax.ShapeDtypeStruct((_B, _DP), jnp.float32),
    scratch_types=[
        pltpu.VMEM((_CTX, _BPW), jnp.int32),
        pltpu.VMEM((2, _CTX * _GRP, _D), jnp.float32),
        pltpu.VMEM((_BPW, _DP), jnp.float32),
        pltpu.SemaphoreType.DMA,
        pltpu.SemaphoreType.DMA,
    ],
    compiler_params=pltpu.CompilerParams(use_tc_tiling_on_sc=False),
)
def _pool(idx_hbm, table_hbm, out_hbm, idx20_v, rows_v, pooled_v,
          sem0, sem1):
    wid = lax.axis_index("s") * _NC + lax.axis_index("c")
    base = wid * _BPW
    for c in range(_CTX):
        pltpu.sync_copy(
            idx_hbm.at[pl.ds(pl.multiple_of(c * _B + base, 8), _BPW)],
            idx20_v.at[c])

    sems = (sem0, sem1)
    zero = jnp.zeros((16,), jnp.float32)

    def fire(g, slot):
        off = pl.multiple_of(g * _GRP, 8)
        for c in range(_CTX):
            pltpu.async_copy(
                table_hbm.at[idx20_v.at[c, pl.ds(off, _GRP)]],
                rows_v.at[slot, pl.ds(c * _GRP, _GRP)],
                sems[slot])

    def drain(slot):
        pltpu.make_async_copy(
            table_hbm.at[pl.ds(0, _CTX * _GRP), :],
            rows_v.at[slot],
            sems[slot]).wait()

    fire(0, 0)
    fire(1, 1)

    def grp2(jj, carry):
        j = jj * 2
        for b in range(2):
            g = j + b
            drain(b)

            def row_body(i, acc_carry, b=b, g=g):
                row = g * _GRP + i
                for k in range(_D // 16):
                    acc = rows_v[b, i, pl.ds(k * 16, 16)]
                    for c in range(1, _CTX):
                        acc = acc + rows_v[b, c * _GRP + i, pl.ds(k * 16, 16)]
                    pooled_v[row, pl.ds(k * 16, 16)] = acc * (1.0 / _CTX)
                for k in range(_D // 16, _DP // 16):
                    pooled_v[row, pl.ds(k * 16, 16)] = zero
                return acc_carry

            lax.fori_loop(0, _GRP, row_body, 0)

            @pl.when(g + 2 < _NGRP)
            def _():
                fire(g + 2, b)
        return carry

    lax.fori_loop(0, _NGRP // 2, grp2, 0)
    pltpu.sync_copy(pooled_v, out_hbm.at[pl.ds(pl.multiple_of(base, 8), _BPW)])


_VB = 1536


def _proj_body(p_ref, w_ref, b_ref, o_ref):
    ot = lax.dot_general(
        w_ref[...], p_ref[...][:, :_D], (((0,), (1,)), ((), ())),
        preferred_element_type=jnp.float32,
    )
    o_ref[...] = ot + b_ref[...].T


@jax.jit
def _project_t(pooled, w_t, bias2d):
    return pl.pallas_call(
        _proj_body,
        grid=(pl.cdiv(_V, _VB),),
        in_specs=[
            pl.BlockSpec((_B, _DP), lambda v: (0, 0)),
            pl.BlockSpec((_D, _VB), lambda v: (0, v)),
            pl.BlockSpec((1, _VB), lambda v: (0, v)),
        ],
        out_specs=pl.BlockSpec((_VB, _B), lambda v: (v, 0)),
        out_shape=jax.ShapeDtypeStruct((_V, _B), jnp.float32),
    )(pooled, w_t, bias2d)


def kernel(context_words_indices, embeddings, linear_w, linear_b):
    idx_flat = jnp.asarray(context_words_indices, jnp.int32).T.reshape(-1)
    pooled128 = _pool(idx_flat, embeddings)
    out_t = _project_t(pooled128, linear_w.T, linear_b.reshape(1, _V))
    return out_t.T


# speedup vs baseline: 1.0346x; 1.0191x over previous
import functools

import jax
import jax.numpy as jnp
from jax import lax
from jax.experimental import pallas as pl
from jax.experimental.pallas import tpu as pltpu
from jax.experimental.pallas import tpu_sc as plsc

_B, _CTX, _D, _V = 4096, 20, 64, 100000
_DP = 128
_NC, _NS = 2, 16
_NW = _NC * _NS
_BPW = _B // _NW
_GRP = 32
_NGRP = _BPW // _GRP

_mesh = plsc.VectorSubcoreMesh(core_axis_name="c", subcore_axis_name="s")


@functools.partial(
    pl.kernel,
    mesh=_mesh,
    out_type=jax.ShapeDtypeStruct((_B, _DP), jnp.float32),
    scratch_types=[
        pltpu.VMEM((_CTX, _BPW), jnp.int32),
        pltpu.VMEM((2, _CTX * _GRP, _D), jnp.float32),
        pltpu.VMEM((_BPW, _DP), jnp.float32),
        pltpu.SemaphoreType.DMA,
        pltpu.SemaphoreType.DMA,
        pltpu.SemaphoreType.DMA,
    ],
    compiler_params=pltpu.CompilerParams(use_tc_tiling_on_sc=False),
)
def _pool(idx_hbm, table_hbm, out_hbm, idx20_v, rows_v, pooled_v,
          sem0, sem1, sem_idx):
    wid = lax.axis_index("s") * _NC + lax.axis_index("c")
    base = wid * _BPW
    for c in range(_CTX):
        pltpu.async_copy(
            idx_hbm.at[pl.ds(pl.multiple_of(c * _B + base, 8), _BPW)],
            idx20_v.at[c], sem_idx)
    for c in range(_CTX):
        pltpu.make_async_copy(
            idx_hbm.at[pl.ds(0, _BPW)], idx20_v.at[c], sem_idx).wait()

    sems = (sem0, sem1)
    zero = jnp.zeros((16,), jnp.float32)

    def fire(g, slot):
        off = pl.multiple_of(g * _GRP, 8)
        for c in range(_CTX):
            pltpu.async_copy(
                table_hbm.at[idx20_v.at[c, pl.ds(off, _GRP)]],
                rows_v.at[slot, pl.ds(c * _GRP, _GRP)],
                sems[slot])

    def drain(slot):
        pltpu.make_async_copy(
            table_hbm.at[pl.ds(0, _CTX * _GRP), :],
            rows_v.at[slot],
            sems[slot]).wait()

    fire(0, 0)
    fire(1, 1)

    def grp2(jj, carry):
        j = jj * 2
        for b in range(2):
            g = j + b
            drain(b)

            def row_body(i, acc_carry, b=b, g=g):
                row = g * _GRP + i
                for k in range(_D // 16):
                    acc = rows_v[b, i, pl.ds(k * 16, 16)]
                    for c in range(1, _CTX):
                        acc = acc + rows_v[b, c * _GRP + i, pl.ds(k * 16, 16)]
                    pooled_v[row, pl.ds(k * 16, 16)] = acc * (1.0 / _CTX)
                for k in range(_D // 16, _DP // 16):
                    pooled_v[row, pl.ds(k * 16, 16)] = zero
                return acc_carry

            lax.fori_loop(0, _GRP, row_body, 0)

            @pl.when(g + 2 < _NGRP)
            def _():
                fire(g + 2, b)
        return carry

    lax.fori_loop(0, _NGRP // 2, grp2, 0)
    pltpu.sync_copy(pooled_v, out_hbm.at[pl.ds(pl.multiple_of(base, 8), _BPW)])


_VB = 1024


def _proj_body(p_ref, w_ref, b_ref, o_ref):
    ot = lax.dot_general(
        w_ref[...], p_ref[...][:, :_D], (((0,), (1,)), ((), ())),
        preferred_element_type=jnp.float32,
    )
    o_ref[...] = ot + b_ref[...].T


@jax.jit
def _project_t(pooled, w_t, bias2d):
    return pl.pallas_call(
        _proj_body,
        grid=(pl.cdiv(_V, _VB),),
        in_specs=[
            pl.BlockSpec((_B, _DP), lambda v: (0, 0)),
            pl.BlockSpec((_D, _VB), lambda v: (0, v)),
            pl.BlockSpec((1, _VB), lambda v: (0, v)),
        ],
        out_specs=pl.BlockSpec((_VB, _B), lambda v: (v, 0)),
        out_shape=jax.ShapeDtypeStruct((_V, _B), jnp.float32),
    )(pooled, w_t, bias2d)


def kernel(context_words_indices, embeddings, linear_w, linear_b):
    idx_flat = jnp.asarray(context_words_indices, jnp.int32).T.reshape(-1)
    pooled128 = _pool(idx_flat, embeddings)
    out_t = _project_t(pooled128, linear_w.T, linear_b.reshape(1, _V))
    return out_t.T
